# baseline (speedup 1.0000x reference)
import functools

import jax
import jax.numpy as jnp
from jax import lax
from jax.experimental import pallas as pl
from jax.experimental.pallas import tpu as pltpu
from jax.experimental.pallas import tpu_sc as plsc

B = 8
N = 2048
S1 = 128
K1NB = 8
BW1 = 0.1
BW2 = 0.2
F32 = jnp.float32


def _fold(L):
    We = L['w'].T * L['g'][None, :]
    be = (L['b'] * L['g'] + L['bt'])[None, :]
    return We, be


def _perm_linear(lw, g, bt, lb, c, o):
    P = lw.reshape(lw.shape[0], c, o)
    P = jnp.transpose(P, (2, 1, 0)).reshape(o * c, lw.shape[0])
    P = P * g[None, :]
    be = (lb * g + bt)[None, :]
    return P, be


def _density_body(xt_ref, x_ref, out_ref, *, bw, n):
    XT = xt_ref[0]
    cn2 = jnp.sum(XT * XT, axis=0, keepdims=True)
    sc = -1.0 / (2.0 * bw * bw)
    out_ref[0, :, 0:3] = x_ref[0]
    chunk = 256 if n >= 256 else n
    for j in range(n // chunk):
        Xc = x_ref[0, j * chunk:(j + 1) * chunk, :]
        rn2 = jnp.sum(Xc * Xc, axis=1, keepdims=True)
        d = rn2 + cn2 - 2.0 * jnp.dot(Xc, XT, preferred_element_type=F32)
        s = jnp.sum(jnp.exp(d * sc), axis=1, keepdims=True)
        dens = s * (1.0 / (2.5 * bw * n))
        out_ref[0, j * chunk:(j + 1) * chunk, 3:4] = 1.0 / dens


def _sigmoid(x):
    return 1.0 / (1.0 + jnp.exp(-x))


def _fps_sc_tile(x0v, x1v, x2v, distv, idxv, out_hbm, b, *, npoint, n):
    nch = n // 16
    iota16 = lax.iota(jnp.int32, 16)
    lane0 = iota16 == 0

    def init(j, _):
        distv[pl.ds(j * 16, 16)] = jnp.full((16,), 1e10, F32)
        return 0
    lax.fori_loop(0, nch, init, 0)

    def body(i, carry):
        far, c0, c1, c2, acc = carry
        accn = jnp.where(iota16 == i % 16, jnp.full((16,), far, jnp.int32),
                         acc)

        @pl.when(i % 16 == 15)
        def _flush():
            idxv[pl.ds(i - 15, 16)] = accn

        def chunk(j, ch):
            rv, ri, r0, r1, r2 = ch
            sl = pl.ds(j * 16, 16)
            v0 = x0v[sl]
            v1 = x1v[sl]
            v2 = x2v[sl]
            t0 = v0 - c0
            t1 = v1 - c1
            t2 = v2 - c2
            d = t0 * t0 + t1 * t1 + t2 * t2
            dn = jnp.minimum(distv[sl], d)
            distv[sl] = dn
            upd = dn > rv
            rv = jnp.where(upd, dn, rv)
            ri = jnp.where(upd, j * 16 + iota16, ri)
            r0 = jnp.where(upd, v0, r0)
            r1 = jnp.where(upd, v1, r1)
            r2 = jnp.where(upd, v2, r2)
            return rv, ri, r0, r1, r2

        z16 = jnp.zeros((16,), F32)
        rv, ri, r0, r1, r2 = lax.fori_loop(
            0, nch, chunk,
            (jnp.full((16,), -1.0, F32), jnp.zeros((16,), jnp.int32),
             z16, z16, z16))
        m = jnp.max(rv)
        far2 = jnp.min(jnp.where(rv == m, ri, n))
        pick = ri == far2
        n0 = jnp.sum(jnp.where(pick, r0, 0.0))
        n1 = jnp.sum(jnp.where(pick, r1, 0.0))
        n2 = jnp.sum(jnp.where(pick, r2, 0.0))
        return far2, n0, n1, n2, accn

    h0 = x0v[pl.ds(0, 16)]
    h1 = x1v[pl.ds(0, 16)]
    h2 = x2v[pl.ds(0, 16)]
    i0 = jnp.sum(jnp.where(lane0, h0, 0.0))
    i1 = jnp.sum(jnp.where(lane0, h1, 0.0))
    i2 = jnp.sum(jnp.where(lane0, h2, 0.0))
    lax.fori_loop(0, npoint, body,
                  (jnp.int32(0), i0, i1, i2, jnp.zeros((16,), jnp.int32)))
    pltpu.sync_copy(idxv, out_hbm.at[pl.ds(b * npoint, npoint)])


def _sa1_body(xt_ref, g_ref, fps_ref, m1wa_ref, m1wb_ref, m1b_ref,
              w1a_ref, w1ab_ref, w1b_ref, w1bb_ref, w1c_ref, w1cb_ref,
              d1a_ref, d1ab_ref, d1b_ref, d1bb_ref, d1c_ref, d1cb_ref,
              lp_ref, lpb_ref, eb_ref, fb_ref, nx_ref, out_ref,
              *, n, s1, knb):
    XT = xt_ref[0]
    G = g_ref[0]
    fps = fps_ref[0]
    col = jax.lax.broadcasted_iota(jnp.int32, (s1, n), 1)

    ohf = (col == fps).astype(F32)
    NG = jnp.dot(ohf, G, preferred_element_type=F32)
    new_xyz = NG[:, 0:3]
    nx_ref[0] = new_xyz

    cn2 = jnp.sum(XT * XT, axis=0, keepdims=True)
    rn2 = jnp.sum(new_xyz * new_xyz, axis=1, keepdims=True)
    sq = (-2.0 * jnp.dot(new_xyz, XT, preferred_element_type=F32)
          + rn2 + cn2)

    iks = []
    d = sq
    for k in range(knb):
        val = jnp.min(d, axis=1, keepdims=True)
        ik = jnp.min(jnp.where(d == val, col, n), axis=1, keepdims=True)
        d = jnp.where(col == ik, 1e30, d)
        iks.append(ik)

    r = knb * s1
    ik_all = jnp.concatenate(iks, axis=0)
    col_all = jax.lax.broadcasted_iota(jnp.int32, (r, n), 1)
    OH = (col_all == ik_all).astype(F32)
    Gk = jnp.dot(OH, G, preferred_element_type=F32)
    nx_all = jnp.concatenate([new_xyz] * knb, axis=0)
    gxn = Gk[:, 0:3] - nx_all
    x = jnp.maximum(jnp.dot(gxn, m1wa_ref[:], preferred_element_type=F32)
                    + jnp.dot(Gk[:, 0:3], m1wb_ref[:],
                              preferred_element_type=F32)
                    + m1b_ref[:], 0.0)
    h = jnp.maximum(jnp.dot(gxn, w1a_ref[:], preferred_element_type=F32)
                    + w1ab_ref[:], 0.0)
    h = jnp.maximum(jnp.dot(h, w1b_ref[:], preferred_element_type=F32)
                    + w1bb_ref[:], 0.0)
    w = jnp.maximum(jnp.dot(h, w1c_ref[:], preferred_element_type=F32)
                    + w1cb_ref[:], 0.0)
    gd = Gk[:, 3:4]

    inv_max = gd[0:s1]
    for k in range(1, knb):
        inv_max = jnp.maximum(inv_max, gd[k * s1:(k + 1) * s1])
    ds0 = gd / jnp.concatenate([inv_max] * knb, axis=0)
    h = jnp.maximum(ds0 * d1a_ref[:] + d1ab_ref[:], 0.0)
    h = jnp.maximum(jnp.dot(h, d1b_ref[:], preferred_element_type=F32)
                    + d1bb_ref[:], 0.0)
    dsc = _sigmoid(jnp.dot(h, d1c_ref[:], preferred_element_type=F32)
                   + d1cb_ref[:])
    x = x * dsc

    Wt = jnp.dot(w, eb_ref[:], preferred_element_type=F32)
    Xt = jnp.dot(x, fb_ref[:], preferred_element_type=F32)
    yk = jnp.dot(Wt * Xt, lp_ref[:], preferred_element_type=F32)
    y = yk[0:s1]
    for k in range(1, knb):
        y = y + yk[k * s1:(k + 1) * s1]
    out_ref[0] = jnp.maximum(y + lpb_ref[:], 0.0)


def _sa2_body(nx_ref, p_ref, m2w_ref, m2b_ref,
              w2a_ref, w2ab_ref, w2b_ref, w2bb_ref, w2c_ref, w2cb_ref,
              d2a_ref, d2ab_ref, d2b_ref, d2bb_ref, d2c_ref, d2cb_ref,
              lp_ref, lpb_ref, eb_ref, fb_ref, out_ref, *, s1, bw):
    X = nx_ref[0]
    P = p_ref[0]

    nt = (((1,), (1,)), ((), ()))
    X2 = X * X
    rn2 = jnp.sum(X2, axis=1, keepdims=True)
    cn2 = jax.lax.dot_general(jnp.ones((1, 3), F32), X2, nt,
                              preferred_element_type=F32)
    sq = (-2.0 * jax.lax.dot_general(X, X, nt, preferred_element_type=F32)
          + rn2 + cn2)
    g = jnp.exp(sq * (-1.0 / (2.0 * bw * bw))) * (1.0 / (2.5 * bw))
    dens = jnp.sum(g, axis=1, keepdims=True) * (1.0 / s1)
    invd = 1.0 / dens
    inv_max = jnp.max(invd)
    ds0 = invd / inv_max
    h = jnp.maximum(ds0 * d2a_ref[:] + d2ab_ref[:], 0.0)
    h = jnp.maximum(jnp.dot(h, d2b_ref[:], preferred_element_type=F32)
                    + d2bb_ref[:], 0.0)
    dsc = _sigmoid(jnp.dot(h, d2c_ref[:], preferred_element_type=F32)
                   + d2cb_ref[:])

    np67 = jnp.concatenate([X, P], axis=1)
    x = jnp.maximum(jnp.dot(np67, m2w_ref[:], preferred_element_type=F32)
                    + m2b_ref[:], 0.0)
    x = x * dsc
    h = jnp.maximum(jnp.dot(X, w2a_ref[:], preferred_element_type=F32)
                    + w2ab_ref[:], 0.0)
    h = jnp.maximum(jnp.dot(h, w2b_ref[:], preferred_element_type=F32)
                    + w2bb_ref[:], 0.0)
    w = jnp.maximum(jnp.dot(h, w2c_ref[:], preferred_element_type=F32)
                    + w2cb_ref[:], 0.0)

    Wt = jnp.dot(w, eb_ref[:], preferred_element_type=F32)
    Xt = jnp.dot(x, fb_ref[:], preferred_element_type=F32)
    zs = jnp.sum(Wt * Xt, axis=0, keepdims=True)
    y = jnp.maximum(jnp.dot(zs, lp_ref[:], preferred_element_type=F32)
                    + lpb_ref[:], 0.0)
    out_ref[0] = y


def _full(shape):
    nd = len(shape)
    return pl.BlockSpec(shape, lambda b: (0,) * nd)


def kernel(xyz, params):
    xyz = xyz.astype(F32)
    xyz_t = jnp.swapaxes(xyz, 1, 2)
    p1, p2 = params['sa1'], params['sa2']

    m1w, m1b = _fold(p1['mlp'][0])
    w1a, w1ab = _fold(p1['weightnet'][0])
    w1b, w1bb = _fold(p1['weightnet'][1])
    w1c, w1cb = _fold(p1['weightnet'][2])
    d1a, d1ab = _fold(p1['density'][0])
    d1b, d1bb = _fold(p1['density'][1])
    d1c, d1cb = _fold(p1['density'][2])
    lp1, lpb1 = _perm_linear(p1['linear_w'], p1['bnl_g'], p1['bnl_b'],
                             p1['linear_b'], 64, 16)

    m2w, m2b = _fold(p2['mlp'][0])
    w2a, w2ab = _fold(p2['weightnet'][0])
    w2b, w2bb = _fold(p2['weightnet'][1])
    w2c, w2cb = _fold(p2['weightnet'][2])
    d2a, d2ab = _fold(p2['density'][0])
    d2b, d2bb = _fold(p2['density'][1])
    d2c, d2cb = _fold(p2['density'][2])
    lp2, lpb2 = _perm_linear(p2['linear_w'], p2['bnl_g'], p2['bnl_b'],
                             p2['linear_b'], 16, 16)

    G = pl.pallas_call(
        functools.partial(_density_body, bw=BW1, n=N),
        grid=(B,),
        in_specs=[pl.BlockSpec((1, 3, N), lambda b: (b, 0, 0)),
                  pl.BlockSpec((1, N, 3), lambda b: (b, 0, 0))],
        out_specs=pl.BlockSpec((1, N, 4), lambda b: (b, 0, 0)),
        out_shape=jax.ShapeDtypeStruct((B, N, 4), F32),
    )(xyz, xyz_t)

    mesh = plsc.VectorSubcoreMesh(core_axis_name="c", subcore_axis_name="s")

    @functools.partial(
        pl.kernel, mesh=mesh,
        out_type=jax.ShapeDtypeStruct((B * S1,), jnp.int32),
        scratch_types=[pltpu.VMEM((N,), F32)] * 4
                      + [pltpu.VMEM((S1,), jnp.int32)],
        compiler_params=pltpu.CompilerParams(needs_layout_passes=False),
    )
    def _fps_sc(xyz_hbm, out_hbm, x0v, x1v, x2v, distv, idxv):
        wid = lax.axis_index("s") * 2 + lax.axis_index("c")

        @pl.when(wid < B)
        def _():
            pltpu.sync_copy(xyz_hbm.at[pl.ds((wid * 3 + 0) * N, N)], x0v)
            pltpu.sync_copy(xyz_hbm.at[pl.ds((wid * 3 + 1) * N, N)], x1v)
            pltpu.sync_copy(xyz_hbm.at[pl.ds((wid * 3 + 2) * N, N)], x2v)
            _fps_sc_tile(x0v, x1v, x2v, distv, idxv, out_hbm, wid,
                         npoint=S1, n=N)

    fps = _fps_sc(xyz.reshape(B * 3 * N)).reshape(B, S1, 1)

    o16 = jnp.arange(16, dtype=jnp.int32)
    E1 = (o16[:, None] == (jnp.arange(1024, dtype=jnp.int32) // 64)[None, :]
          ).astype(F32)
    F1 = (jnp.arange(64, dtype=jnp.int32)[:, None]
          == (jnp.arange(1024, dtype=jnp.int32) % 64)[None, :]).astype(F32)
    E2 = (o16[:, None] == (jnp.arange(256, dtype=jnp.int32) // 16)[None, :]
          ).astype(F32)
    F2 = (o16[:, None] == (jnp.arange(256, dtype=jnp.int32) % 16)[None, :]
          ).astype(F32)
    wargs = [m1w[0:3], m1w[3:6], m1b, w1a, w1ab, w1b, w1bb, w1c, w1cb,
             d1a, d1ab, d1b, d1bb, d1c, d1cb, lp1, lpb1, E1, F1]
    new_xyz, l1p = pl.pallas_call(
        functools.partial(_sa1_body, n=N, s1=S1, knb=K1NB),
        grid=(B,),
        in_specs=[pl.BlockSpec((1, 3, N), lambda b: (b, 0, 0)),
                  pl.BlockSpec((1, N, 4), lambda b: (b, 0, 0)),
                  pl.BlockSpec((1, S1, 1), lambda b: (b, 0, 0))]
                 + [_full(w.shape) for w in wargs],
        out_specs=[pl.BlockSpec((1, S1, 3), lambda b: (b, 0, 0)),
                   pl.BlockSpec((1, S1, 64), lambda b: (b, 0, 0))],
        out_shape=[jax.ShapeDtypeStruct((B, S1, 3), F32),
                   jax.ShapeDtypeStruct((B, S1, 64), F32)],
    )(xyz, G, fps, *wargs)

    wargs2 = [m2w, m2b, w2a, w2ab, w2b, w2bb, w2c, w2cb,
              d2a, d2ab, d2b, d2bb, d2c, d2cb, lp2, lpb2, E2, F2]
    out = pl.pallas_call(
        functools.partial(_sa2_body, s1=S1, bw=BW2),
        grid=(B,),
        in_specs=[pl.BlockSpec((1, S1, 3), lambda b: (b, 0, 0)),
                  pl.BlockSpec((1, S1, 64), lambda b: (b, 0, 0))]
                 + [_full(w.shape) for w in wargs2],
        out_specs=pl.BlockSpec((1, 1, 16), lambda b: (b, 0, 0)),
        out_shape=jax.ShapeDtypeStruct((B, 1, 16), F32),
    )(new_xyz, l1p, *wargs2)

    return out.reshape(B, 16)


# speedup vs baseline: 1.5743x; 1.5743x over previous
import functools

import jax
import jax.numpy as jnp
from jax import lax
from jax.experimental import pallas as pl
from jax.experimental.pallas import tpu as pltpu
from jax.experimental.pallas import tpu_sc as plsc

B = 8
N = 2048
S1 = 128
K1NB = 8
BW1 = 0.1
BW2 = 0.2
F32 = jnp.float32


def _fold(L):
    We = L['w'].T * L['g'][None, :]
    be = (L['b'] * L['g'] + L['bt'])[None, :]
    return We, be


def _perm_linear(lw, g, bt, lb, c, o):
    P = lw.reshape(lw.shape[0], c, o)
    P = jnp.transpose(P, (2, 1, 0)).reshape(o * c, lw.shape[0])
    P = P * g[None, :]
    be = (lb * g + bt)[None, :]
    return P, be


def _density_body(xt_ref, x_ref, out_ref, *, bw, n):
    XT = xt_ref[0]
    cn2 = jnp.sum(XT * XT, axis=0, keepdims=True)
    sc = -1.0 / (2.0 * bw * bw)
    out_ref[0, :, 0:3] = x_ref[0]
    chunk = 256 if n >= 256 else n
    for j in range(n // chunk):
        Xc = x_ref[0, j * chunk:(j + 1) * chunk, :]
        rn2 = jnp.sum(Xc * Xc, axis=1, keepdims=True)
        d = rn2 + cn2 - 2.0 * jnp.dot(Xc, XT, preferred_element_type=F32)
        s = jnp.sum(jnp.exp(d * sc), axis=1, keepdims=True)
        dens = s * (1.0 / (2.5 * bw * n))
        out_ref[0, j * chunk:(j + 1) * chunk, 3:4] = 1.0 / dens


def _sigmoid(x):
    return 1.0 / (1.0 + jnp.exp(-x))


def _fps_sc_tile(x0v, x1v, x2v, distv, idxv, out_hbm, b, *, npoint, n):
    nch = n // 16
    iota16 = lax.iota(jnp.int32, 16)
    lane0 = iota16 == 0

    @plsc.parallel_loop(0, nch, unroll=4)
    def _init(j):
        distv[pl.ds(j * 16, 16)] = jnp.full((16,), 1e10, F32)

    def body(i, carry):
        far, c0, c1, c2, acc = carry
        accn = jnp.where(iota16 == i % 16, jnp.full((16,), far, jnp.int32),
                         acc)

        @pl.when(i % 16 == 15)
        def _flush():
            idxv[pl.ds(i - 15, 16)] = accn

        @plsc.parallel_loop(0, nch, unroll=4,
                            carry=(jnp.full((16,), -1.0, F32),
                                   jnp.zeros((16,), jnp.int32)))
        def chunk(j, ch):
            rv, ri = ch
            sl = pl.ds(j * 16, 16)
            t0 = x0v[sl] - c0
            t1 = x1v[sl] - c1
            t2 = x2v[sl] - c2
            d = t0 * t0 + t1 * t1 + t2 * t2
            dn = jnp.minimum(distv[sl], d)
            distv[sl] = dn
            upd = dn > rv
            return jnp.where(upd, dn, rv), jnp.where(upd, j * 16 + iota16, ri)

        rv, ri = chunk
        m = jnp.max(rv)
        far2 = jnp.min(jnp.where(rv == m, ri, n))
        base = (far2 // 16) * 16
        lm = iota16 == far2 - base
        n0 = jnp.sum(jnp.where(lm, x0v[pl.ds(base, 16)], 0.0))
        n1 = jnp.sum(jnp.where(lm, x1v[pl.ds(base, 16)], 0.0))
        n2 = jnp.sum(jnp.where(lm, x2v[pl.ds(base, 16)], 0.0))
        return far2, n0, n1, n2, accn

    h0 = x0v[pl.ds(0, 16)]
    h1 = x1v[pl.ds(0, 16)]
    h2 = x2v[pl.ds(0, 16)]
    i0 = jnp.sum(jnp.where(lane0, h0, 0.0))
    i1 = jnp.sum(jnp.where(lane0, h1, 0.0))
    i2 = jnp.sum(jnp.where(lane0, h2, 0.0))
    lax.fori_loop(0, npoint, body,
                  (jnp.int32(0), i0, i1, i2, jnp.zeros((16,), jnp.int32)))
    pltpu.sync_copy(idxv, out_hbm.at[pl.ds(b * npoint, npoint)])


def _sa1_body(xt_ref, g_ref, fps_ref, m1wa_ref, m1wb_ref, m1b_ref,
              w1a_ref, w1ab_ref, w1b_ref, w1bb_ref, w1c_ref, w1cb_ref,
              d1a_ref, d1ab_ref, d1b_ref, d1bb_ref, d1c_ref, d1cb_ref,
              lp_ref, lpb_ref, eb_ref, fb_ref, nx_ref, out_ref,
              *, n, s1, knb):
    XT = xt_ref[0]
    G = g_ref[0]
    fps = fps_ref[0]
    col = jax.lax.broadcasted_iota(jnp.int32, (s1, n), 1)

    ohf = (col == fps).astype(F32)
    NG = jnp.dot(ohf, G, preferred_element_type=F32)
    new_xyz = NG[:, 0:3]
    nx_ref[0] = new_xyz

    cn2 = jnp.sum(XT * XT, axis=0, keepdims=True)
    rn2 = jnp.sum(new_xyz * new_xyz, axis=1, keepdims=True)
    sq = (-2.0 * jnp.dot(new_xyz, XT, preferred_element_type=F32)
          + rn2 + cn2)

    iks = []
    d = sq
    for k in range(knb):
        val = jnp.min(d, axis=1, keepdims=True)
        ik = jnp.min(jnp.where(d == val, col, n), axis=1, keepdims=True)
        d = jnp.where(col == ik, 1e30, d)
        iks.append(ik)

    r = knb * s1
    ik_all = jnp.concatenate(iks, axis=0)
    col_all = jax.lax.broadcasted_iota(jnp.int32, (r, n), 1)
    OH = (col_all == ik_all).astype(F32)
    Gk = jnp.dot(OH, G, preferred_element_type=F32)
    nx_all = jnp.concatenate([new_xyz] * knb, axis=0)
    gxn = Gk[:, 0:3] - nx_all
    x = jnp.maximum(jnp.dot(gxn, m1wa_ref[:], preferred_element_type=F32)
                    + jnp.dot(Gk[:, 0:3], m1wb_ref[:],
                              preferred_element_type=F32)
                    + m1b_ref[:], 0.0)
    h = jnp.maximum(jnp.dot(gxn, w1a_ref[:], preferred_element_type=F32)
                    + w1ab_ref[:], 0.0)
    h = jnp.maximum(jnp.dot(h, w1b_ref[:], preferred_element_type=F32)
                    + w1bb_ref[:], 0.0)
    w = jnp.maximum(jnp.dot(h, w1c_ref[:], preferred_element_type=F32)
                    + w1cb_ref[:], 0.0)
    gd = Gk[:, 3:4]

    inv_max = gd[0:s1]
    for k in range(1, knb):
        inv_max = jnp.maximum(inv_max, gd[k * s1:(k + 1) * s1])
    ds0 = gd / jnp.concatenate([inv_max] * knb, axis=0)
    h = jnp.maximum(ds0 * d1a_ref[:] + d1ab_ref[:], 0.0)
    h = jnp.maximum(jnp.dot(h, d1b_ref[:], preferred_element_type=F32)
                    + d1bb_ref[:], 0.0)
    dsc = _sigmoid(jnp.dot(h, d1c_ref[:], preferred_element_type=F32)
                   + d1cb_ref[:])
    x = x * dsc

    Wt = jnp.dot(w, eb_ref[:], preferred_element_type=F32)
    Xt = jnp.dot(x, fb_ref[:], preferred_element_type=F32)
    yk = jnp.dot(Wt * Xt, lp_ref[:], preferred_element_type=F32)
    y = yk[0:s1]
    for k in range(1, knb):
        y = y + yk[k * s1:(k + 1) * s1]
    out_ref[0] = jnp.maximum(y + lpb_ref[:], 0.0)


def _sa2_body(nx_ref, p_ref, m2w_ref, m2b_ref,
              w2a_ref, w2ab_ref, w2b_ref, w2bb_ref, w2c_ref, w2cb_ref,
              d2a_ref, d2ab_ref, d2b_ref, d2bb_ref, d2c_ref, d2cb_ref,
              lp_ref, lpb_ref, eb_ref, fb_ref, out_ref, *, s1, bw):
    X = nx_ref[0]
    P = p_ref[0]

    nt = (((1,), (1,)), ((), ()))
    X2 = X * X
    rn2 = jnp.sum(X2, axis=1, keepdims=True)
    cn2 = jax.lax.dot_general(jnp.ones((1, 3), F32), X2, nt,
                              preferred_element_type=F32)
    sq = (-2.0 * jax.lax.dot_general(X, X, nt, preferred_element_type=F32)
          + rn2 + cn2)
    g = jnp.exp(sq * (-1.0 / (2.0 * bw * bw))) * (1.0 / (2.5 * bw))
    dens = jnp.sum(g, axis=1, keepdims=True) * (1.0 / s1)
    invd = 1.0 / dens
    inv_max = jnp.max(invd)
    ds0 = invd / inv_max
    h = jnp.maximum(ds0 * d2a_ref[:] + d2ab_ref[:], 0.0)
    h = jnp.maximum(jnp.dot(h, d2b_ref[:], preferred_element_type=F32)
                    + d2bb_ref[:], 0.0)
    dsc = _sigmoid(jnp.dot(h, d2c_ref[:], preferred_element_type=F32)
                   + d2cb_ref[:])

    np67 = jnp.concatenate([X, P], axis=1)
    x = jnp.maximum(jnp.dot(np67, m2w_ref[:], preferred_element_type=F32)
                    + m2b_ref[:], 0.0)
    x = x * dsc
    h = jnp.maximum(jnp.dot(X, w2a_ref[:], preferred_element_type=F32)
                    + w2ab_ref[:], 0.0)
    h = jnp.maximum(jnp.dot(h, w2b_ref[:], preferred_element_type=F32)
                    + w2bb_ref[:], 0.0)
    w = jnp.maximum(jnp.dot(h, w2c_ref[:], preferred_element_type=F32)
                    + w2cb_ref[:], 0.0)

    Wt = jnp.dot(w, eb_ref[:], preferred_element_type=F32)
    Xt = jnp.dot(x, fb_ref[:], preferred_element_type=F32)
    zs = jnp.sum(Wt * Xt, axis=0, keepdims=True)
    y = jnp.maximum(jnp.dot(zs, lp_ref[:], preferred_element_type=F32)
                    + lpb_ref[:], 0.0)
    out_ref[0] = y


def _full(shape):
    nd = len(shape)
    return pl.BlockSpec(shape, lambda b: (0,) * nd)


def kernel(xyz, params):
    xyz = xyz.astype(F32)
    xyz_t = jnp.swapaxes(xyz, 1, 2)
    p1, p2 = params['sa1'], params['sa2']

    m1w, m1b = _fold(p1['mlp'][0])
    w1a, w1ab = _fold(p1['weightnet'][0])
    w1b, w1bb = _fold(p1['weightnet'][1])
    w1c, w1cb = _fold(p1['weightnet'][2])
    d1a, d1ab = _fold(p1['density'][0])
    d1b, d1bb = _fold(p1['density'][1])
    d1c, d1cb = _fold(p1['density'][2])
    lp1, lpb1 = _perm_linear(p1['linear_w'], p1['bnl_g'], p1['bnl_b'],
                             p1['linear_b'], 64, 16)

    m2w, m2b = _fold(p2['mlp'][0])
    w2a, w2ab = _fold(p2['weightnet'][0])
    w2b, w2bb = _fold(p2['weightnet'][1])
    w2c, w2cb = _fold(p2['weightnet'][2])
    d2a, d2ab = _fold(p2['density'][0])
    d2b, d2bb = _fold(p2['density'][1])
    d2c, d2cb = _fold(p2['density'][2])
    lp2, lpb2 = _perm_linear(p2['linear_w'], p2['bnl_g'], p2['bnl_b'],
                             p2['linear_b'], 16, 16)

    G = pl.pallas_call(
        functools.partial(_density_body, bw=BW1, n=N),
        grid=(B,),
        in_specs=[pl.BlockSpec((1, 3, N), lambda b: (b, 0, 0)),
                  pl.BlockSpec((1, N, 3), lambda b: (b, 0, 0))],
        out_specs=pl.BlockSpec((1, N, 4), lambda b: (b, 0, 0)),
        out_shape=jax.ShapeDtypeStruct((B, N, 4), F32),
    )(xyz, xyz_t)

    mesh = plsc.VectorSubcoreMesh(core_axis_name="c", subcore_axis_name="s")

    @functools.partial(
        pl.kernel, mesh=mesh,
        out_type=jax.ShapeDtypeStruct((B * S1,), jnp.int32),
        scratch_types=[pltpu.VMEM((N,), F32)] * 4
                      + [pltpu.VMEM((S1,), jnp.int32)],
        compiler_params=pltpu.CompilerParams(needs_layout_passes=False),
    )
    def _fps_sc(xyz_hbm, out_hbm, x0v, x1v, x2v, distv, idxv):
        wid = lax.axis_index("s") * 2 + lax.axis_index("c")

        @pl.when(wid < B)
        def _():
            pltpu.sync_copy(xyz_hbm.at[pl.ds((wid * 3 + 0) * N, N)], x0v)
            pltpu.sync_copy(xyz_hbm.at[pl.ds((wid * 3 + 1) * N, N)], x1v)
            pltpu.sync_copy(xyz_hbm.at[pl.ds((wid * 3 + 2) * N, N)], x2v)
            _fps_sc_tile(x0v, x1v, x2v, distv, idxv, out_hbm, wid,
                         npoint=S1, n=N)

    fps = _fps_sc(xyz.reshape(B * 3 * N)).reshape(B, S1, 1)

    o16 = jnp.arange(16, dtype=jnp.int32)
    E1 = (o16[:, None] == (jnp.arange(1024, dtype=jnp.int32) // 64)[None, :]
          ).astype(F32)
    F1 = (jnp.arange(64, dtype=jnp.int32)[:, None]
          == (jnp.arange(1024, dtype=jnp.int32) % 64)[None, :]).astype(F32)
    E2 = (o16[:, None] == (jnp.arange(256, dtype=jnp.int32) // 16)[None, :]
          ).astype(F32)
    F2 = (o16[:, None] == (jnp.arange(256, dtype=jnp.int32) % 16)[None, :]
          ).astype(F32)
    wargs = [m1w[0:3], m1w[3:6], m1b, w1a, w1ab, w1b, w1bb, w1c, w1cb,
             d1a, d1ab, d1b, d1bb, d1c, d1cb, lp1, lpb1, E1, F1]
    new_xyz, l1p = pl.pallas_call(
        functools.partial(_sa1_body, n=N, s1=S1, knb=K1NB),
        grid=(B,),
        in_specs=[pl.BlockSpec((1, 3, N), lambda b: (b, 0, 0)),
                  pl.BlockSpec((1, N, 4), lambda b: (b, 0, 0)),
                  pl.BlockSpec((1, S1, 1), lambda b: (b, 0, 0))]
                 + [_full(w.shape) for w in wargs],
        out_specs=[pl.BlockSpec((1, S1, 3), lambda b: (b, 0, 0)),
                   pl.BlockSpec((1, S1, 64), lambda b: (b, 0, 0))],
        out_shape=[jax.ShapeDtypeStruct((B, S1, 3), F32),
                   jax.ShapeDtypeStruct((B, S1, 64), F32)],
    )(xyz, G, fps, *wargs)

    wargs2 = [m2w, m2b, w2a, w2ab, w2b, w2bb, w2c, w2cb,
              d2a, d2ab, d2b, d2bb, d2c, d2cb, lp2, lpb2, E2, F2]
    out = pl.pallas_call(
        functools.partial(_sa2_body, s1=S1, bw=BW2),
        grid=(B,),
        in_specs=[pl.BlockSpec((1, S1, 3), lambda b: (b, 0, 0)),
                  pl.BlockSpec((1, S1, 64), lambda b: (b, 0, 0))]
                 + [_full(w.shape) for w in wargs2],
        out_specs=pl.BlockSpec((1, 1, 16), lambda b: (b, 0, 0)),
        out_shape=jax.ShapeDtypeStruct((B, 1, 16), F32),
    )(new_xyz, l1p, *wargs2)

    return out.reshape(B, 16)


# speedup vs baseline: 1.5750x; 1.0005x over previous
import functools

import jax
import jax.numpy as jnp
from jax import lax
from jax.experimental import pallas as pl
from jax.experimental.pallas import tpu as pltpu
from jax.experimental.pallas import tpu_sc as plsc

B = 8
N = 2048
S1 = 128
K1NB = 8
BW1 = 0.1
BW2 = 0.2
F32 = jnp.float32


def _fold(L):
    We = L['w'].T * L['g'][None, :]
    be = (L['b'] * L['g'] + L['bt'])[None, :]
    return We, be


def _perm_linear(lw, g, bt, lb, c, o):
    P = lw.reshape(lw.shape[0], c, o)
    P = jnp.transpose(P, (2, 1, 0)).reshape(o * c, lw.shape[0])
    P = P * g[None, :]
    be = (lb * g + bt)[None, :]
    return P, be


def _density_body(xt_ref, x_ref, out_ref, *, bw, n):
    XT = xt_ref[0]
    cn2 = jnp.sum(XT * XT, axis=0, keepdims=True)
    sc = -1.0 / (2.0 * bw * bw)
    out_ref[0, :, 0:3] = x_ref[0]
    chunk = 256 if n >= 256 else n
    for j in range(n // chunk):
        Xc = x_ref[0, j * chunk:(j + 1) * chunk, :]
        rn2 = jnp.sum(Xc * Xc, axis=1, keepdims=True)
        d = rn2 + cn2 - 2.0 * jnp.dot(Xc, XT, preferred_element_type=F32)
        s = jnp.sum(jnp.exp(d * sc), axis=1, keepdims=True)
        dens = s * (1.0 / (2.5 * bw * n))
        out_ref[0, j * chunk:(j + 1) * chunk, 3:4] = 1.0 / dens


def _sigmoid(x):
    return 1.0 / (1.0 + jnp.exp(-x))


def _fps_sc_tile(x0v, x1v, x2v, distv, idxv, out_hbm, b, *, npoint, n):
    nch = n // 16
    iota16 = lax.iota(jnp.int32, 16)
    lane0 = iota16 == 0

    @plsc.parallel_loop(0, nch, unroll=8)
    def _init(j):
        distv[pl.ds(j * 16, 16)] = jnp.full((16,), 1e10, F32)

    def body(i, carry):
        far, c0, c1, c2, acc = carry
        accn = jnp.where(iota16 == i % 16, jnp.full((16,), far, jnp.int32),
                         acc)

        @pl.when(i % 16 == 15)
        def _flush():
            idxv[pl.ds(i - 15, 16)] = accn

        @plsc.parallel_loop(0, nch, unroll=8,
                            carry=(jnp.full((16,), -1.0, F32),
                                   jnp.zeros((16,), jnp.int32)))
        def chunk(j, ch):
            rv, ri = ch
            sl = pl.ds(j * 16, 16)
            t0 = x0v[sl] - c0
            t1 = x1v[sl] - c1
            t2 = x2v[sl] - c2
            d = t0 * t0 + t1 * t1 + t2 * t2
            dn = jnp.minimum(distv[sl], d)
            distv[sl] = dn
            upd = dn > rv
            return jnp.where(upd, dn, rv), jnp.where(upd, j * 16 + iota16, ri)

        rv, ri = chunk
        m = jnp.max(rv)
        far2 = jnp.min(jnp.where(rv == m, ri, n))
        base = (far2 // 16) * 16
        lm = iota16 == far2 - base
        n0 = jnp.sum(jnp.where(lm, x0v[pl.ds(base, 16)], 0.0))
        n1 = jnp.sum(jnp.where(lm, x1v[pl.ds(base, 16)], 0.0))
        n2 = jnp.sum(jnp.where(lm, x2v[pl.ds(base, 16)], 0.0))
        return far2, n0, n1, n2, accn

    h0 = x0v[pl.ds(0, 16)]
    h1 = x1v[pl.ds(0, 16)]
    h2 = x2v[pl.ds(0, 16)]
    i0 = jnp.sum(jnp.where(lane0, h0, 0.0))
    i1 = jnp.sum(jnp.where(lane0, h1, 0.0))
    i2 = jnp.sum(jnp.where(lane0, h2, 0.0))
    lax.fori_loop(0, npoint, body,
                  (jnp.int32(0), i0, i1, i2, jnp.zeros((16,), jnp.int32)))
    pltpu.sync_copy(idxv, out_hbm.at[pl.ds(b * npoint, npoint)])


def _sa1_body(xt_ref, g_ref, fps_ref, m1wa_ref, m1wb_ref, m1b_ref,
              w1a_ref, w1ab_ref, w1b_ref, w1bb_ref, w1c_ref, w1cb_ref,
              d1a_ref, d1ab_ref, d1b_ref, d1bb_ref, d1c_ref, d1cb_ref,
              lp_ref, lpb_ref, eb_ref, fb_ref, nx_ref, out_ref,
              *, n, s1, knb):
    XT = xt_ref[0]
    G = g_ref[0]
    fps = fps_ref[0]
    col = jax.lax.broadcasted_iota(jnp.int32, (s1, n), 1)

    ohf = (col == fps).astype(F32)
    NG = jnp.dot(ohf, G, preferred_element_type=F32)
    new_xyz = NG[:, 0:3]
    nx_ref[0] = new_xyz

    cn2 = jnp.sum(XT * XT, axis=0, keepdims=True)
    rn2 = jnp.sum(new_xyz * new_xyz, axis=1, keepdims=True)
    sq = (-2.0 * jnp.dot(new_xyz, XT, preferred_element_type=F32)
          + rn2 + cn2)

    iks = []
    d = sq
    for k in range(knb):
        val = jnp.min(d, axis=1, keepdims=True)
        ik = jnp.min(jnp.where(d == val, col, n), axis=1, keepdims=True)
        d = jnp.where(col == ik, 1e30, d)
        iks.append(ik)

    r = knb * s1
    ik_all = jnp.concatenate(iks, axis=0)
    col_all = jax.lax.broadcasted_iota(jnp.int32, (r, n), 1)
    OH = (col_all == ik_all).astype(F32)
    Gk = jnp.dot(OH, G, preferred_element_type=F32)
    nx_all = jnp.concatenate([new_xyz] * knb, axis=0)
    gxn = Gk[:, 0:3] - nx_all
    x = jnp.maximum(jnp.dot(gxn, m1wa_ref[:], preferred_element_type=F32)
                    + jnp.dot(Gk[:, 0:3], m1wb_ref[:],
                              preferred_element_type=F32)
                    + m1b_ref[:], 0.0)
    h = jnp.maximum(jnp.dot(gxn, w1a_ref[:], preferred_element_type=F32)
                    + w1ab_ref[:], 0.0)
    h = jnp.maximum(jnp.dot(h, w1b_ref[:], preferred_element_type=F32)
                    + w1bb_ref[:], 0.0)
    w = jnp.maximum(jnp.dot(h, w1c_ref[:], preferred_element_type=F32)
                    + w1cb_ref[:], 0.0)
    gd = Gk[:, 3:4]

    inv_max = gd[0:s1]
    for k in range(1, knb):
        inv_max = jnp.maximum(inv_max, gd[k * s1:(k + 1) * s1])
    ds0 = gd / jnp.concatenate([inv_max] * knb, axis=0)
    h = jnp.maximum(ds0 * d1a_ref[:] + d1ab_ref[:], 0.0)
    h = jnp.maximum(jnp.dot(h, d1b_ref[:], preferred_element_type=F32)
                    + d1bb_ref[:], 0.0)
    dsc = _sigmoid(jnp.dot(h, d1c_ref[:], preferred_element_type=F32)
                   + d1cb_ref[:])
    x = x * dsc

    Wt = jnp.dot(w, eb_ref[:], preferred_element_type=F32)
    Xt = jnp.dot(x, fb_ref[:], preferred_element_type=F32)
    yk = jnp.dot(Wt * Xt, lp_ref[:], preferred_element_type=F32)
    y = yk[0:s1]
    for k in range(1, knb):
        y = y + yk[k * s1:(k + 1) * s1]
    out_ref[0] = jnp.maximum(y + lpb_ref[:], 0.0)


def _sa2_body(nx_ref, p_ref, m2w_ref, m2b_ref,
              w2a_ref, w2ab_ref, w2b_ref, w2bb_ref, w2c_ref, w2cb_ref,
              d2a_ref, d2ab_ref, d2b_ref, d2bb_ref, d2c_ref, d2cb_ref,
              lp_ref, lpb_ref, eb_ref, fb_ref, out_ref, *, s1, bw):
    X = nx_ref[0]
    P = p_ref[0]

    nt = (((1,), (1,)), ((), ()))
    X2 = X * X
    rn2 = jnp.sum(X2, axis=1, keepdims=True)
    cn2 = jax.lax.dot_general(jnp.ones((1, 3), F32), X2, nt,
                              preferred_element_type=F32)
    sq = (-2.0 * jax.lax.dot_general(X, X, nt, preferred_element_type=F32)
          + rn2 + cn2)
    g = jnp.exp(sq * (-1.0 / (2.0 * bw * bw))) * (1.0 / (2.5 * bw))
    dens = jnp.sum(g, axis=1, keepdims=True) * (1.0 / s1)
    invd = 1.0 / dens
    inv_max = jnp.max(invd)
    ds0 = invd / inv_max
    h = jnp.maximum(ds0 * d2a_ref[:] + d2ab_ref[:], 0.0)
    h = jnp.maximum(jnp.dot(h, d2b_ref[:], preferred_element_type=F32)
                    + d2bb_ref[:], 0.0)
    dsc = _sigmoid(jnp.dot(h, d2c_ref[:], preferred_element_type=F32)
                   + d2cb_ref[:])

    np67 = jnp.concatenate([X, P], axis=1)
    x = jnp.maximum(jnp.dot(np67, m2w_ref[:], preferred_element_type=F32)
                    + m2b_ref[:], 0.0)
    x = x * dsc
    h = jnp.maximum(jnp.dot(X, w2a_ref[:], preferred_element_type=F32)
                    + w2ab_ref[:], 0.0)
    h = jnp.maximum(jnp.dot(h, w2b_ref[:], preferred_element_type=F32)
                    + w2bb_ref[:], 0.0)
    w = jnp.maximum(jnp.dot(h, w2c_ref[:], preferred_element_type=F32)
                    + w2cb_ref[:], 0.0)

    Wt = jnp.dot(w, eb_ref[:], preferred_element_type=F32)
    Xt = jnp.dot(x, fb_ref[:], preferred_element_type=F32)
    zs = jnp.sum(Wt * Xt, axis=0, keepdims=True)
    y = jnp.maximum(jnp.dot(zs, lp_ref[:], preferred_element_type=F32)
                    + lpb_ref[:], 0.0)
    out_ref[0] = y


def _full(shape):
    nd = len(shape)
    return pl.BlockSpec(shape, lambda b: (0,) * nd)


def kernel(xyz, params):
    xyz = xyz.astype(F32)
    xyz_t = jnp.swapaxes(xyz, 1, 2)
    p1, p2 = params['sa1'], params['sa2']

    m1w, m1b = _fold(p1['mlp'][0])
    w1a, w1ab = _fold(p1['weightnet'][0])
    w1b, w1bb = _fold(p1['weightnet'][1])
    w1c, w1cb = _fold(p1['weightnet'][2])
    d1a, d1ab = _fold(p1['density'][0])
    d1b, d1bb = _fold(p1['density'][1])
    d1c, d1cb = _fold(p1['density'][2])
    lp1, lpb1 = _perm_linear(p1['linear_w'], p1['bnl_g'], p1['bnl_b'],
                             p1['linear_b'], 64, 16)

    m2w, m2b = _fold(p2['mlp'][0])
    w2a, w2ab = _fold(p2['weightnet'][0])
    w2b, w2bb = _fold(p2['weightnet'][1])
    w2c, w2cb = _fold(p2['weightnet'][2])
    d2a, d2ab = _fold(p2['density'][0])
    d2b, d2bb = _fold(p2['density'][1])
    d2c, d2cb = _fold(p2['density'][2])
    lp2, lpb2 = _perm_linear(p2['linear_w'], p2['bnl_g'], p2['bnl_b'],
                             p2['linear_b'], 16, 16)

    G = pl.pallas_call(
        functools.partial(_density_body, bw=BW1, n=N),
        grid=(B,),
        in_specs=[pl.BlockSpec((1, 3, N), lambda b: (b, 0, 0)),
                  pl.BlockSpec((1, N, 3), lambda b: (b, 0, 0))],
        out_specs=pl.BlockSpec((1, N, 4), lambda b: (b, 0, 0)),
        out_shape=jax.ShapeDtypeStruct((B, N, 4), F32),
    )(xyz, xyz_t)

    mesh = plsc.VectorSubcoreMesh(core_axis_name="c", subcore_axis_name="s")

    @functools.partial(
        pl.kernel, mesh=mesh,
        out_type=jax.ShapeDtypeStruct((B * S1,), jnp.int32),
        scratch_types=[pltpu.VMEM((N,), F32)] * 4
                      + [pltpu.VMEM((S1,), jnp.int32)],
        compiler_params=pltpu.CompilerParams(needs_layout_passes=False),
    )
    def _fps_sc(xyz_hbm, out_hbm, x0v, x1v, x2v, distv, idxv):
        wid = lax.axis_index("s") * 2 + lax.axis_index("c")

        @pl.when(wid < B)
        def _():
            pltpu.sync_copy(xyz_hbm.at[pl.ds((wid * 3 + 0) * N, N)], x0v)
            pltpu.sync_copy(xyz_hbm.at[pl.ds((wid * 3 + 1) * N, N)], x1v)
            pltpu.sync_copy(xyz_hbm.at[pl.ds((wid * 3 + 2) * N, N)], x2v)
            _fps_sc_tile(x0v, x1v, x2v, distv, idxv, out_hbm, wid,
                         npoint=S1, n=N)

    fps = _fps_sc(xyz.reshape(B * 3 * N)).reshape(B, S1, 1)

    o16 = jnp.arange(16, dtype=jnp.int32)
    E1 = (o16[:, None] == (jnp.arange(1024, dtype=jnp.int32) // 64)[None, :]
          ).astype(F32)
    F1 = (jnp.arange(64, dtype=jnp.int32)[:, None]
          == (jnp.arange(1024, dtype=jnp.int32) % 64)[None, :]).astype(F32)
    E2 = (o16[:, None] == (jnp.arange(256, dtype=jnp.int32) // 16)[None, :]
          ).astype(F32)
    F2 = (o16[:, None] == (jnp.arange(256, dtype=jnp.int32) % 16)[None, :]
          ).astype(F32)
    wargs = [m1w[0:3], m1w[3:6], m1b, w1a, w1ab, w1b, w1bb, w1c, w1cb,
             d1a, d1ab, d1b, d1bb, d1c, d1cb, lp1, lpb1, E1, F1]
    new_xyz, l1p = pl.pallas_call(
        functools.partial(_sa1_body, n=N, s1=S1, knb=K1NB),
        grid=(B,),
        in_specs=[pl.BlockSpec((1, 3, N), lambda b: (b, 0, 0)),
                  pl.BlockSpec((1, N, 4), lambda b: (b, 0, 0)),
                  pl.BlockSpec((1, S1, 1), lambda b: (b, 0, 0))]
                 + [_full(w.shape) for w in wargs],
        out_specs=[pl.BlockSpec((1, S1, 3), lambda b: (b, 0, 0)),
                   pl.BlockSpec((1, S1, 64), lambda b: (b, 0, 0))],
        out_shape=[jax.ShapeDtypeStruct((B, S1, 3), F32),
                   jax.ShapeDtypeStruct((B, S1, 64), F32)],
    )(xyz, G, fps, *wargs)

    wargs2 = [m2w, m2b, w2a, w2ab, w2b, w2bb, w2c, w2cb,
              d2a, d2ab, d2b, d2bb, d2c, d2cb, lp2, lpb2, E2, F2]
    out = pl.pallas_call(
        functools.partial(_sa2_body, s1=S1, bw=BW2),
        grid=(B,),
        in_specs=[pl.BlockSpec((1, S1, 3), lambda b: (b, 0, 0)),
                  pl.BlockSpec((1, S1, 64), lambda b: (b, 0, 0))]
                 + [_full(w.shape) for w in wargs2],
        out_specs=pl.BlockSpec((1, 1, 16), lambda b: (b, 0, 0)),
        out_shape=jax.ShapeDtypeStruct((B, 1, 16), F32),
    )(new_xyz, l1p, *wargs2)

    return out.reshape(B, 16)


# speedup vs baseline: 1.6147x; 1.0252x over previous
import functools

import jax
import jax.numpy as jnp
from jax import lax
from jax.experimental import pallas as pl
from jax.experimental.pallas import tpu as pltpu
from jax.experimental.pallas import tpu_sc as plsc

B = 8
N = 2048
S1 = 128
K1NB = 8
BW1 = 0.1
BW2 = 0.2
F32 = jnp.float32


def _fold(L):
    We = L['w'].T * L['g'][None, :]
    be = (L['b'] * L['g'] + L['bt'])[None, :]
    return We, be


def _perm_linear(lw, g, bt, lb, c, o):
    P = lw.reshape(lw.shape[0], c, o)
    P = jnp.transpose(P, (2, 1, 0)).reshape(o * c, lw.shape[0])
    P = P * g[None, :]
    be = (lb * g + bt)[None, :]
    return P, be


def _density_body(xt_ref, x_ref, out_ref, *, bw, n):
    XT = xt_ref[0]
    cn2 = jnp.sum(XT * XT, axis=0, keepdims=True)
    sc = -1.0 / (2.0 * bw * bw)
    out_ref[0, :, 0:3] = x_ref[0]
    chunk = 256 if n >= 256 else n
    for j in range(n // chunk):
        Xc = x_ref[0, j * chunk:(j + 1) * chunk, :]
        rn2 = jnp.sum(Xc * Xc, axis=1, keepdims=True)
        d = rn2 + cn2 - 2.0 * jnp.dot(Xc, XT, preferred_element_type=F32)
        s = jnp.sum(jnp.exp(d * sc), axis=1, keepdims=True)
        dens = s * (1.0 / (2.5 * bw * n))
        out_ref[0, j * chunk:(j + 1) * chunk, 3:4] = 1.0 / dens


def _sigmoid(x):
    return 1.0 / (1.0 + jnp.exp(-x))


def _fps_sc_tile(x0v, x1v, x2v, distv, idxv, out_hbm, b, *, npoint, n):
    nch = n // 16
    iota16 = lax.iota(jnp.int32, 16)
    lane0 = iota16 == 0

    @plsc.parallel_loop(0, nch, unroll=8)
    def _init(j):
        distv[pl.ds(j * 16, 16)] = jnp.full((16,), 1e10, F32)

    def body(i, carry):
        far, c0, c1, c2, acc = carry
        accn = jnp.where(iota16 == i % 16, jnp.full((16,), far, jnp.int32),
                         acc)

        @pl.when(i % 16 == 15)
        def _flush():
            idxv[pl.ds(i - 15, 16)] = accn

        @plsc.parallel_loop(0, nch, unroll=8,
                            carry=(jnp.full((16,), -1.0, F32),
                                   jnp.zeros((16,), jnp.int32)))
        def chunk(j, ch):
            rv, ri = ch
            sl = pl.ds(j * 16, 16)
            t0 = x0v[sl] - c0
            t1 = x1v[sl] - c1
            t2 = x2v[sl] - c2
            d = t0 * t0 + t1 * t1 + t2 * t2
            dn = jnp.minimum(distv[sl], d)
            distv[sl] = dn
            upd = dn > rv
            return jnp.where(upd, dn, rv), jnp.where(upd, j * 16 + iota16, ri)

        rv, ri = chunk
        m = jnp.max(rv)
        far2 = jnp.min(jnp.where(rv == m, ri, n))
        base = (far2 // 16) * 16
        lm = iota16 == far2 - base
        n0 = jnp.sum(jnp.where(lm, x0v[pl.ds(base, 16)], 0.0))
        n1 = jnp.sum(jnp.where(lm, x1v[pl.ds(base, 16)], 0.0))
        n2 = jnp.sum(jnp.where(lm, x2v[pl.ds(base, 16)], 0.0))
        return far2, n0, n1, n2, accn

    h0 = x0v[pl.ds(0, 16)]
    h1 = x1v[pl.ds(0, 16)]
    h2 = x2v[pl.ds(0, 16)]
    i0 = jnp.sum(jnp.where(lane0, h0, 0.0))
    i1 = jnp.sum(jnp.where(lane0, h1, 0.0))
    i2 = jnp.sum(jnp.where(lane0, h2, 0.0))
    lax.fori_loop(0, npoint, body,
                  (jnp.int32(0), i0, i1, i2, jnp.zeros((16,), jnp.int32)))
    pltpu.sync_copy(idxv, out_hbm.at[pl.ds(b * npoint, npoint)])


def _sa_body(xt_ref, g_ref, fps_ref, m1wa_ref, m1wb_ref, m1b_ref,
             w1a_ref, w1ab_ref, w1b_ref, w1bb_ref, w1c_ref, w1cb_ref,
             d1a_ref, d1ab_ref, d1b_ref, d1bb_ref, d1c_ref, d1cb_ref,
             lp_ref, lpb_ref, eb_ref, fb_ref,
             m2w_ref, m2b_ref,
             w2a_ref, w2ab_ref, w2b_ref, w2bb_ref, w2c_ref, w2cb_ref,
             d2a_ref, d2ab_ref, d2b_ref, d2bb_ref, d2c_ref, d2cb_ref,
             lp2_ref, lpb2_ref, eb2_ref, fb2_ref, out_ref,
             *, n, s1, knb, bw2):
    XT = xt_ref[0]
    G = g_ref[0]
    fps = fps_ref[0]
    col = jax.lax.broadcasted_iota(jnp.int32, (s1, n), 1)

    ohf = (col == fps).astype(F32)
    NG = jnp.dot(ohf, G, preferred_element_type=F32)
    new_xyz = NG[:, 0:3]

    cn2 = jnp.sum(XT * XT, axis=0, keepdims=True)
    rn2 = jnp.sum(new_xyz * new_xyz, axis=1, keepdims=True)
    sq = (-2.0 * jnp.dot(new_xyz, XT, preferred_element_type=F32)
          + rn2 + cn2)

    iks = []
    d = sq
    for k in range(knb):
        val = jnp.min(d, axis=1, keepdims=True)
        ik = jnp.min(jnp.where(d == val, col, n), axis=1, keepdims=True)
        d = jnp.where(col == ik, 1e30, d)
        iks.append(ik)

    r = knb * s1
    ik_all = jnp.concatenate(iks, axis=0)
    col_all = jax.lax.broadcasted_iota(jnp.int32, (r, n), 1)
    OH = (col_all == ik_all).astype(F32)
    Gk = jnp.dot(OH, G, preferred_element_type=F32)
    nx_all = jnp.concatenate([new_xyz] * knb, axis=0)
    gxn = Gk[:, 0:3] - nx_all
    x = jnp.maximum(jnp.dot(gxn, m1wa_ref[:], preferred_element_type=F32)
                    + jnp.dot(Gk[:, 0:3], m1wb_ref[:],
                              preferred_element_type=F32)
                    + m1b_ref[:], 0.0)
    h = jnp.maximum(jnp.dot(gxn, w1a_ref[:], preferred_element_type=F32)
                    + w1ab_ref[:], 0.0)
    h = jnp.maximum(jnp.dot(h, w1b_ref[:], preferred_element_type=F32)
                    + w1bb_ref[:], 0.0)
    w = jnp.maximum(jnp.dot(h, w1c_ref[:], preferred_element_type=F32)
                    + w1cb_ref[:], 0.0)
    gd = Gk[:, 3:4]

    inv_max = gd[0:s1]
    for k in range(1, knb):
        inv_max = jnp.maximum(inv_max, gd[k * s1:(k + 1) * s1])
    ds0 = gd / jnp.concatenate([inv_max] * knb, axis=0)
    h = jnp.maximum(ds0 * d1a_ref[:] + d1ab_ref[:], 0.0)
    h = jnp.maximum(jnp.dot(h, d1b_ref[:], preferred_element_type=F32)
                    + d1bb_ref[:], 0.0)
    dsc = _sigmoid(jnp.dot(h, d1c_ref[:], preferred_element_type=F32)
                   + d1cb_ref[:])
    x = x * dsc

    Wt = jnp.dot(w, eb_ref[:], preferred_element_type=F32)
    Xt = jnp.dot(x, fb_ref[:], preferred_element_type=F32)
    yk = jnp.dot(Wt * Xt, lp_ref[:], preferred_element_type=F32)
    y = yk[0:s1]
    for k in range(1, knb):
        y = y + yk[k * s1:(k + 1) * s1]
    P = jnp.maximum(y + lpb_ref[:], 0.0)

    X = new_xyz
    nt = (((1,), (1,)), ((), ()))
    X2 = X * X
    rn2b = jnp.sum(X2, axis=1, keepdims=True)
    cn2b = jax.lax.dot_general(jnp.ones((1, 3), F32), X2, nt,
                               preferred_element_type=F32)
    sqb = (-2.0 * jax.lax.dot_general(X, X, nt, preferred_element_type=F32)
           + rn2b + cn2b)
    gb = jnp.exp(sqb * (-1.0 / (2.0 * bw2 * bw2))) * (1.0 / (2.5 * bw2))
    densb = jnp.sum(gb, axis=1, keepdims=True) * (1.0 / s1)
    invdb = 1.0 / densb
    inv_maxb = jnp.max(invdb)
    ds0b = invdb / inv_maxb
    hb = jnp.maximum(ds0b * d2a_ref[:] + d2ab_ref[:], 0.0)
    hb = jnp.maximum(jnp.dot(hb, d2b_ref[:], preferred_element_type=F32)
                     + d2bb_ref[:], 0.0)
    dscb = _sigmoid(jnp.dot(hb, d2c_ref[:], preferred_element_type=F32)
                    + d2cb_ref[:])

    xb = jnp.maximum(jnp.dot(X, m2w_ref[0:3], preferred_element_type=F32)
                     + jnp.dot(P, m2w_ref[3:67], preferred_element_type=F32)
                     + m2b_ref[:], 0.0)
    xb = xb * dscb
    hb = jnp.maximum(jnp.dot(X, w2a_ref[:], preferred_element_type=F32)
                     + w2ab_ref[:], 0.0)
    hb = jnp.maximum(jnp.dot(hb, w2b_ref[:], preferred_element_type=F32)
                     + w2bb_ref[:], 0.0)
    wb = jnp.maximum(jnp.dot(hb, w2c_ref[:], preferred_element_type=F32)
                     + w2cb_ref[:], 0.0)

    Wtb = jnp.dot(wb, eb2_ref[:], preferred_element_type=F32)
    Xtb = jnp.dot(xb, fb2_ref[:], preferred_element_type=F32)
    zsb = jnp.sum(Wtb * Xtb, axis=0, keepdims=True)
    yb = jnp.maximum(jnp.dot(zsb, lp2_ref[:], preferred_element_type=F32)
                     + lpb2_ref[:], 0.0)
    out_ref[0] = yb


def _full(shape):
    nd = len(shape)
    return pl.BlockSpec(shape, lambda b: (0,) * nd)


def kernel(xyz, params):
    xyz = xyz.astype(F32)
    xyz_t = jnp.swapaxes(xyz, 1, 2)
    p1, p2 = params['sa1'], params['sa2']

    m1w, m1b = _fold(p1['mlp'][0])
    w1a, w1ab = _fold(p1['weightnet'][0])
    w1b, w1bb = _fold(p1['weightnet'][1])
    w1c, w1cb = _fold(p1['weightnet'][2])
    d1a, d1ab = _fold(p1['density'][0])
    d1b, d1bb = _fold(p1['density'][1])
    d1c, d1cb = _fold(p1['density'][2])
    lp1, lpb1 = _perm_linear(p1['linear_w'], p1['bnl_g'], p1['bnl_b'],
                             p1['linear_b'], 64, 16)

    m2w, m2b = _fold(p2['mlp'][0])
    w2a, w2ab = _fold(p2['weightnet'][0])
    w2b, w2bb = _fold(p2['weightnet'][1])
    w2c, w2cb = _fold(p2['weightnet'][2])
    d2a, d2ab = _fold(p2['density'][0])
    d2b, d2bb = _fold(p2['density'][1])
    d2c, d2cb = _fold(p2['density'][2])
    lp2, lpb2 = _perm_linear(p2['linear_w'], p2['bnl_g'], p2['bnl_b'],
                             p2['linear_b'], 16, 16)

    G = pl.pallas_call(
        functools.partial(_density_body, bw=BW1, n=N),
        grid=(B,),
        in_specs=[pl.BlockSpec((1, 3, N), lambda b: (b, 0, 0)),
                  pl.BlockSpec((1, N, 3), lambda b: (b, 0, 0))],
        out_specs=pl.BlockSpec((1, N, 4), lambda b: (b, 0, 0)),
        out_shape=jax.ShapeDtypeStruct((B, N, 4), F32),
    )(xyz, xyz_t)

    mesh = plsc.VectorSubcoreMesh(core_axis_name="c", subcore_axis_name="s")

    @functools.partial(
        pl.kernel, mesh=mesh,
        out_type=jax.ShapeDtypeStruct((B * S1,), jnp.int32),
        scratch_types=[pltpu.VMEM((N,), F32)] * 4
                      + [pltpu.VMEM((S1,), jnp.int32)],
        compiler_params=pltpu.CompilerParams(needs_layout_passes=False),
    )
    def _fps_sc(xyz_hbm, out_hbm, x0v, x1v, x2v, distv, idxv):
        wid = lax.axis_index("s") * 2 + lax.axis_index("c")

        @pl.when(wid < B)
        def _():
            pltpu.sync_copy(xyz_hbm.at[pl.ds((wid * 3 + 0) * N, N)], x0v)
            pltpu.sync_copy(xyz_hbm.at[pl.ds((wid * 3 + 1) * N, N)], x1v)
            pltpu.sync_copy(xyz_hbm.at[pl.ds((wid * 3 + 2) * N, N)], x2v)
            _fps_sc_tile(x0v, x1v, x2v, distv, idxv, out_hbm, wid,
                         npoint=S1, n=N)

    fps = _fps_sc(xyz.reshape(B * 3 * N)).reshape(B, S1, 1)

    o16 = jnp.arange(16, dtype=jnp.int32)
    E1 = (o16[:, None] == (jnp.arange(1024, dtype=jnp.int32) // 64)[None, :]
          ).astype(F32)
    F1 = (jnp.arange(64, dtype=jnp.int32)[:, None]
          == (jnp.arange(1024, dtype=jnp.int32) % 64)[None, :]).astype(F32)
    E2 = (o16[:, None] == (jnp.arange(256, dtype=jnp.int32) // 16)[None, :]
          ).astype(F32)
    F2 = (o16[:, None] == (jnp.arange(256, dtype=jnp.int32) % 16)[None, :]
          ).astype(F32)
    wargs = [m1w[0:3], m1w[3:6], m1b, w1a, w1ab, w1b, w1bb, w1c, w1cb,
             d1a, d1ab, d1b, d1bb, d1c, d1cb, lp1, lpb1, E1, F1,
             m2w, m2b, w2a, w2ab, w2b, w2bb, w2c, w2cb,
             d2a, d2ab, d2b, d2bb, d2c, d2cb, lp2, lpb2, E2, F2]
    out = pl.pallas_call(
        functools.partial(_sa_body, n=N, s1=S1, knb=K1NB, bw2=BW2),
        grid=(B,),
        in_specs=[pl.BlockSpec((1, 3, N), lambda b: (b, 0, 0)),
                  pl.BlockSpec((1, N, 4), lambda b: (b, 0, 0)),
                  pl.BlockSpec((1, S1, 1), lambda b: (b, 0, 0))]
                 + [_full(w.shape) for w in wargs],
        out_specs=pl.BlockSpec((1, 1, 16), lambda b: (b, 0, 0)),
        out_shape=jax.ShapeDtypeStruct((B, 1, 16), F32),
    )(xyz, G, fps, *wargs)

    return out.reshape(B, 16)
